# baseline (speedup 1.0000x reference)
import functools
import jax
import jax.numpy as jnp
from jax import lax
from jax.experimental import pallas as pl
from jax.experimental.pallas import tpu as pltpu
from jax.experimental.pallas import tpu_sc as plsc

DIM = 32
NODES = 1000000
GRAN = 16
NGRAN_COL = NODES // GRAN
BATCH = 16384
NC = 2
NS = 16
NW = NC * NS
BPW = BATCH // NW
NCHUNK = 4
CHUNK = BPW // NCHUNK
GROUPS = CHUNK // 16


def _rsqrt(x):
    i = plsc.bitcast(x, jnp.int32)
    i = jnp.int32(0x5F3759DF) - (i >> 1)
    y = plsc.bitcast(i, jnp.float32)
    for _ in range(3):
        y = y * (1.5 - 0.5 * x * y * y)
    return y


def _body(u_idx_hbm, v_idx_hbm, tab_hbm, out_hbm,
          uidx_v, vidx_v, ugran_v, vgran_v,
          uix0, uix1, vix0, vix1, ust0, ust1, vst0, vst1, out_v,
          su0, su1, sv0, sv1):
    wid = lax.axis_index("s") * NC + lax.axis_index("c")

    pltpu.sync_copy(u_idx_hbm.at[pl.ds(wid * NCHUNK, NCHUNK)], uidx_v)
    pltpu.sync_copy(v_idx_hbm.at[pl.ds(wid * NCHUNK, NCHUNK)], vidx_v)

    def pregran(k, carry):
        j = k // (CHUNK // 16)
        sl = pl.ds((k % (CHUNK // 16)) * 16, 16)
        ugran_v.at[j][sl] = uidx_v.at[j][sl] >> 4
        vgran_v.at[j][sl] = vidx_v.at[j][sl] >> 4
        return carry

    lax.fori_loop(0, NCHUNK * (CHUNK // 16), pregran, 0)

    lane = lax.iota(jnp.int32, 16)
    slots = ((uix0, ust0, su0, vix0, vst0, sv0),
             (uix1, ust1, su1, vix1, vst1, sv1))

    def enq(j, c, slot):
        uix, ust, su, vix, vst, sv = slot
        col = tab_hbm.at[c]
        pltpu.async_copy(col.at[ugran_v.at[j]], ust, su)
        pltpu.async_copy(col.at[vgran_v.at[j]], vst, sv)

    dummy = tab_hbm.at[0].at[pl.ds(0, CHUNK), :]

    for j in range(NCHUNK):
        enq(j, 0, slots[0])
        enq(j, 1, slots[1])

        def step(cc, acc):
            for half in range(2):
                uix, ust, su, vix, vst, sv = slots[half]
                c = 2 * cc + half
                pltpu.make_async_copy(dummy, ust, su).wait()
                pltpu.make_async_copy(dummy, vst, sv).wait()
                new = []
                for g in range(GROUPS):
                    sl = pl.ds(g * 16, 16)
                    pvec = lane + g * 16
                    ud = plsc.load_gather(ust, [pvec, uidx_v.at[j][sl] & 15])
                    vd = plsc.load_gather(vst, [pvec, vidx_v.at[j][sl] & 15])
                    uu, vv, uv = acc[3 * g], acc[3 * g + 1], acc[3 * g + 2]
                    new += [uu + ud * ud, vv + vd * vd, uv + ud * vd]
                acc = tuple(new)

                @pl.when(c + 2 < DIM)
                def _():
                    enq(j, c + 2, slots[half])
            return acc

        zeros = tuple(jnp.zeros((16,), jnp.float32) for _ in range(3 * GROUPS))
        acc = lax.fori_loop(0, DIM // 2, step, zeros)

        for g in range(GROUPS):
            uu, vv, uv = acc[3 * g], acc[3 * g + 1], acc[3 * g + 2]
            a = 1.0 - 2.0 * uv + vv
            b = 1.0 - uu
            numsq = a * a * uu - 2.0 * a * b * uv + b * b * vv
            den = jnp.maximum(1.0 - 2.0 * uv + uu * vv, 1e-15)
            n2 = jnp.maximum(numsq / (den * den), 1e-30)
            norm = n2 * _rsqrt(n2)
            arg = jnp.minimum(norm, 1.0 - 1e-5)
            t = arg * arg
            dist = 2.0 * arg * (1.0 + t * (1.0 / 3.0 + t * (1.0 / 5.0
                                + t * (1.0 / 7.0 + t * (1.0 / 9.0)))))
            out_v[pl.ds(j * CHUNK + g * 16, 16)] = dist

    pltpu.sync_copy(out_v, out_hbm.at[pl.ds(wid * BPW, BPW)])


@jax.jit
def _run(u_idx2, v_idx2, tab2):
    mesh = plsc.VectorSubcoreMesh(core_axis_name="c", subcore_axis_name="s")
    f = pl.kernel(
        _body,
        mesh=mesh,
        out_type=jax.ShapeDtypeStruct((BATCH,), jnp.float32),
        scratch_types=[
            pltpu.VMEM((NCHUNK, CHUNK), jnp.int32),
            pltpu.VMEM((NCHUNK, CHUNK), jnp.int32),
            pltpu.VMEM((NCHUNK, CHUNK), jnp.int32),
            pltpu.VMEM((NCHUNK, CHUNK), jnp.int32),
            pltpu.VMEM((CHUNK,), jnp.int32),
            pltpu.VMEM((CHUNK,), jnp.int32),
            pltpu.VMEM((CHUNK,), jnp.int32),
            pltpu.VMEM((CHUNK,), jnp.int32),
            pltpu.VMEM((CHUNK, GRAN), jnp.float32),
            pltpu.VMEM((CHUNK, GRAN), jnp.float32),
            pltpu.VMEM((CHUNK, GRAN), jnp.float32),
            pltpu.VMEM((CHUNK, GRAN), jnp.float32),
            pltpu.VMEM((BPW,), jnp.float32),
            pltpu.SemaphoreType.DMA,
            pltpu.SemaphoreType.DMA,
            pltpu.SemaphoreType.DMA,
            pltpu.SemaphoreType.DMA,
        ],
        compiler_params=pltpu.CompilerParams(
            use_tc_tiling_on_sc=False, needs_layout_passes=False),
    )
    return f(u_idx2, v_idx2, tab2)


def kernel(u_idx, v_idx, embeddings):
    u2 = u_idx.reshape(NW * NCHUNK, CHUNK)
    v2 = v_idx.reshape(NW * NCHUNK, CHUNK)
    tab2 = embeddings.T.reshape(DIM, NODES // GRAN, GRAN)
    return _run(u2, v2, tab2)


# speedup vs baseline: 5.6410x; 5.6410x over previous
import functools
import jax
import jax.numpy as jnp
from jax import lax
from jax.experimental import pallas as pl
from jax.experimental.pallas import tpu as pltpu
from jax.experimental.pallas import tpu_sc as plsc

DIM = 32
BATCH = 16384
NC = 2
NS = 16
NW = NC * NS
BPW = BATCH // NW
NCHUNK = 4
CHUNK = BPW // NCHUNK
NGROUP = BPW // 16


def _rsqrt(x):
    i = plsc.bitcast(x, jnp.int32)
    i = jnp.int32(0x5F3759DF) - (i >> 1)
    y = plsc.bitcast(i, jnp.float32)
    for _ in range(3):
        y = y * (1.5 - 0.5 * x * y * y)
    return y


def _body(u_idx_hbm, v_idx_hbm, table_hbm, out_hbm,
          uidx_v, vidx_v, urows_v, vrows_v, out_v, sem):
    wid = lax.axis_index("s") * NC + lax.axis_index("c")

    pltpu.sync_copy(u_idx_hbm.at[pl.ds(wid * NCHUNK, NCHUNK)], uidx_v)
    pltpu.sync_copy(v_idx_hbm.at[pl.ds(wid * NCHUNK, NCHUNK)], vidx_v)

    copies = []
    for j in range(NCHUNK):
        copies.append(pltpu.async_copy(
            table_hbm.at[uidx_v.at[j]],
            urows_v.at[pl.ds(j * CHUNK, CHUNK), :], sem))
        copies.append(pltpu.async_copy(
            table_hbm.at[vidx_v.at[j]],
            vrows_v.at[pl.ds(j * CHUNK, CHUNK), :], sem))
    for cp in copies:
        cp.wait()

    lane = lax.iota(jnp.int32, 16)

    def group(g, carry):
        pvec = lane + g * 16
        uu = jnp.zeros((16,), jnp.float32)
        vv = jnp.zeros((16,), jnp.float32)
        uv = jnp.zeros((16,), jnp.float32)
        for d in range(DIM):
            dvec = jnp.full((16,), d, jnp.int32)
            ud = plsc.load_gather(urows_v, [pvec, dvec])
            vd = plsc.load_gather(vrows_v, [pvec, dvec])
            uu = uu + ud * ud
            vv = vv + vd * vd
            uv = uv + ud * vd

        a = 1.0 - 2.0 * uv + vv
        b = 1.0 - uu
        numsq = a * a * uu - 2.0 * a * b * uv + b * b * vv
        den = jnp.maximum(1.0 - 2.0 * uv + uu * vv, 1e-15)
        n2 = jnp.maximum(numsq / (den * den), 1e-30)
        norm = n2 * _rsqrt(n2)
        arg = jnp.minimum(norm, 1.0 - 1e-5)
        t = arg * arg
        dist = 2.0 * arg * (1.0 + t * (1.0 / 3.0 + t * (1.0 / 5.0
                            + t * (1.0 / 7.0 + t * (1.0 / 9.0)))))
        out_v[pl.ds(g * 16, 16)] = dist
        return carry

    lax.fori_loop(0, NGROUP, group, 0)

    pltpu.sync_copy(out_v, out_hbm.at[pl.ds(wid * BPW, BPW)])


@jax.jit
def _run(u_idx2, v_idx2, embeddings):
    mesh = plsc.VectorSubcoreMesh(core_axis_name="c", subcore_axis_name="s")
    f = pl.kernel(
        _body,
        mesh=mesh,
        out_type=jax.ShapeDtypeStruct((BATCH,), jnp.float32),
        scratch_types=[
            pltpu.VMEM((NCHUNK, CHUNK), jnp.int32),
            pltpu.VMEM((NCHUNK, CHUNK), jnp.int32),
            pltpu.VMEM((BPW, DIM), jnp.float32),
            pltpu.VMEM((BPW, DIM), jnp.float32),
            pltpu.VMEM((BPW,), jnp.float32),
            pltpu.SemaphoreType.DMA,
        ],
        compiler_params=pltpu.CompilerParams(
            use_tc_tiling_on_sc=False, needs_layout_passes=False),
    )
    return f(u_idx2, v_idx2, embeddings)


def kernel(u_idx, v_idx, embeddings):
    u2 = u_idx.reshape(NW * NCHUNK, CHUNK)
    v2 = v_idx.reshape(NW * NCHUNK, CHUNK)
    return _run(u2, v2, embeddings)
